# Initial kernel scaffold; baseline (speedup 1.0000x reference)
import functools

import jax
import jax.numpy as jnp
from jax import lax
from jax.experimental import pallas as pl
from jax.experimental.pallas import tpu as pltpu
from jax.experimental.pallas import tpu_sc as plsc

N = 10000
E = 160000
D = 256
DH = 128
DO = 128

NC = 2
NS = 16
LANES = 16

CH = 128
T = 79
EPS = NS * CH * T
NACC = 10240
STRIPE = NACC // NS

_mesh = functools.partial(
    plsc.VectorSubcoreMesh, core_axis_name="c", subcore_axis_name="s",
    num_cores=NC, num_subcores=NS)


def _deg_body(edges_hbm, out_hbm, idx_v, ones_v, zeros_v, hist):
    cid = lax.axis_index("c")
    sid = lax.axis_index("s")

    def fill(i, _):
        ones_v[pl.ds(i * LANES, LANES)] = jnp.ones((LANES,), jnp.float32)
        return _
    lax.fori_loop(0, CH // LANES, fill, None)

    def zfill(i, _):
        zeros_v[pl.ds(i * LANES, LANES)] = jnp.zeros((LANES,), jnp.float32)
        return _
    lax.fori_loop(0, STRIPE // LANES, zfill, None)

    pltpu.sync_copy(zeros_v, hist.at[pl.ds(sid * STRIPE, STRIPE)])
    plsc.subcore_barrier()

    def chunk(t, _):
        base = sid * (CH * T) + t * CH
        pltpu.sync_copy(edges_hbm.at[cid, pl.ds(base, CH)], idx_v)
        pltpu.sync_copy(ones_v, hist.at[idx_v], add=True)
        return _
    lax.fori_loop(0, T, chunk, None)

    plsc.subcore_barrier()
    pltpu.sync_copy(hist.at[pl.ds(sid * STRIPE, STRIPE)],
                    out_hbm.at[cid, pl.ds(sid * STRIPE, STRIPE)])


def _sc_degrees(edges):
    k = pl.kernel(
        _deg_body,
        out_type=jax.ShapeDtypeStruct((NC, NACC), jnp.float32),
        mesh=_mesh(),
        scratch_types=[
            pltpu.VMEM((CH,), jnp.int32),
            pltpu.VMEM((CH,), jnp.float32),
            pltpu.VMEM((STRIPE,), jnp.float32),
            pltpu.VMEM_SHARED((NACC,), jnp.float32),
        ],
    )
    return k(edges)


def _agg_body(x_hbm, src_hbm, dst_hbm, zeros_hbm, out_hbm,
              sidx, didx, rows, acc, sem):
    cid = lax.axis_index("c")
    sid = lax.axis_index("s")

    pltpu.sync_copy(zeros_hbm, acc.at[pl.ds(sid * STRIPE, STRIPE)])
    plsc.subcore_barrier()

    def chunk(t, _):
        base = sid * (CH * T) + t * CH
        pltpu.sync_copy(src_hbm.at[pl.ds(base, CH)], sidx)
        pltpu.sync_copy(dst_hbm.at[pl.ds(base, CH)], didx)
        pltpu.async_copy(x_hbm.at[cid].at[sidx], rows, sem).wait()
        pltpu.sync_copy(rows, acc.at[didx], add=True)
        return _
    lax.fori_loop(0, T, chunk, None)

    plsc.subcore_barrier()
    pltpu.sync_copy(acc.at[pl.ds(sid * STRIPE, STRIPE)],
                    out_hbm.at[cid, pl.ds(sid * STRIPE, STRIPE)])


def _sc_aggregate(x_split, src_pad, dst_pad, zeros_hbm):
    k = pl.kernel(
        _agg_body,
        out_type=jax.ShapeDtypeStruct((NC, NACC, DH), jnp.float32),
        mesh=_mesh(),
        scratch_types=[
            pltpu.VMEM((CH,), jnp.int32),
            pltpu.VMEM((CH,), jnp.int32),
            pltpu.VMEM((CH, DH), jnp.float32),
            pltpu.VMEM_SHARED((NACC, DH), jnp.float32),
            pltpu.SemaphoreType.DMA,
        ],
    )
    return k(x_split, src_pad, dst_pad, zeros_hbm)


BR = 1000
GRID = N // BR


def _scale_body(feat_ref, odeg_ref, out_ref):
    ns = lax.rsqrt(jnp.maximum(odeg_ref[...], 1.0))
    x = feat_ref[...] * ns
    out_ref[0] = x[:, :DH]
    out_ref[1] = x[:, DH:]


def _tc_scale(feat, odeg):
    return pl.pallas_call(
        _scale_body,
        grid=(GRID,),
        in_specs=[
            pl.BlockSpec((BR, D), lambda i: (i, 0)),
            pl.BlockSpec((BR, 1), lambda i: (i, 0)),
        ],
        out_specs=pl.BlockSpec((NC, BR, DH), lambda i: (0, i, 0)),
        out_shape=jax.ShapeDtypeStruct((NC, N, DH), jnp.float32),
    )(feat, odeg)


def _layer1_body(a_ref, w_ref, b_ref, odeg_ref, ideg_ref, out_ref):
    acc = (jnp.dot(a_ref[0], w_ref[:DH, :], preferred_element_type=jnp.float32)
           + jnp.dot(a_ref[1], w_ref[DH:, :], preferred_element_type=jnp.float32))
    nd = lax.rsqrt(jnp.maximum(ideg_ref[...], 1.0))
    h = jnp.maximum(acc * nd + b_ref[...], 0.0)
    ns = lax.rsqrt(jnp.maximum(odeg_ref[...], 1.0))
    x2 = h * ns
    out_ref[0] = x2[:, :DH]
    out_ref[1] = x2[:, DH:]


def _tc_layer1(a1, w1, b1, odeg, ideg):
    return pl.pallas_call(
        _layer1_body,
        grid=(GRID,),
        in_specs=[
            pl.BlockSpec((NC, BR, DH), lambda i: (0, i, 0)),
            pl.BlockSpec((D, D), lambda i: (0, 0)),
            pl.BlockSpec((1, D), lambda i: (0, 0)),
            pl.BlockSpec((BR, 1), lambda i: (i, 0)),
            pl.BlockSpec((BR, 1), lambda i: (i, 0)),
        ],
        out_specs=pl.BlockSpec((NC, BR, DH), lambda i: (0, i, 0)),
        out_shape=jax.ShapeDtypeStruct((NC, N, DH), jnp.float32),
    )(a1, w1, b1, odeg, ideg)


def _head_body(a_ref, w2_ref, b2_ref, w3_ref, b3_ref, ideg_ref, noise_ref,
               out_ref):
    a0 = a_ref[0]
    a1 = a_ref[1]
    mu = (jnp.dot(a0, w2_ref[:DH, :], preferred_element_type=jnp.float32)
          + jnp.dot(a1, w2_ref[DH:, :], preferred_element_type=jnp.float32))
    ls = (jnp.dot(a0, w3_ref[:DH, :], preferred_element_type=jnp.float32)
          + jnp.dot(a1, w3_ref[DH:, :], preferred_element_type=jnp.float32))
    nd = lax.rsqrt(jnp.maximum(ideg_ref[...], 1.0))
    mu = mu * nd + b2_ref[...]
    ls = ls * nd + b3_ref[...]
    out_ref[...] = mu + noise_ref[...] * jnp.exp(ls)


def _tc_head(a2, w2, b2, w3, b3, ideg, noise):
    return pl.pallas_call(
        _head_body,
        grid=(GRID,),
        in_specs=[
            pl.BlockSpec((NC, BR, DH), lambda i: (0, i, 0)),
            pl.BlockSpec((D, DO), lambda i: (0, 0)),
            pl.BlockSpec((1, DO), lambda i: (0, 0)),
            pl.BlockSpec((D, DO), lambda i: (0, 0)),
            pl.BlockSpec((1, DO), lambda i: (0, 0)),
            pl.BlockSpec((BR, 1), lambda i: (i, 0)),
            pl.BlockSpec((BR, DO), lambda i: (i, 0)),
        ],
        out_specs=pl.BlockSpec((BR, DO), lambda i: (i, 0)),
        out_shape=jax.ShapeDtypeStruct((N, DO), jnp.float32),
    )(a2, w2, b2, w3, b3, ideg, noise)


@jax.jit
def kernel(feat, edge_index, W1, b1, W2, b2, W3, b3, noise):
    src = edge_index[0]
    dst = edge_index[1]
    pad = EPS - E
    sentinel = jnp.full((pad,), N, jnp.int32)
    src_gather = jnp.concatenate([src, jnp.zeros((pad,), jnp.int32)])
    dst_pad = jnp.concatenate([dst, sentinel])
    edges_deg = jnp.stack([jnp.concatenate([src, sentinel]), dst_pad])
    zeros_hbm = jnp.zeros((STRIPE, DH), jnp.float32)

    degs = _sc_degrees(edges_deg)
    odeg = degs[0, :N].reshape(N, 1)
    ideg = degs[1, :N].reshape(N, 1)

    x1 = _tc_scale(feat, odeg)
    a1 = _sc_aggregate(x1, src_gather, dst_pad, zeros_hbm)
    x2 = _tc_layer1(a1, W1, b1.reshape(1, D), odeg, ideg)
    a2 = _sc_aggregate(x2, W1 := None or x2 and None, dst_pad, zeros_hbm)
    return _tc_head(a2, W2, b2.reshape(1, DO), W3, b3.reshape(1, DO),
                    ideg, noise)


# speedup vs baseline: 3.6012x; 3.6012x over previous
import functools

import jax
import jax.numpy as jnp
from jax import lax
from jax.experimental import pallas as pl
from jax.experimental.pallas import tpu as pltpu
from jax.experimental.pallas import tpu_sc as plsc

N = 10000
E = 160000
D = 256
DH = 128
DO = 128

NC = 2
NS = 16
LANES = 16

CH = 128
T = 79
EPS = NS * CH * T
NACC = 10240
STRIPE = NACC // NS

_mesh = functools.partial(
    plsc.VectorSubcoreMesh, core_axis_name="c", subcore_axis_name="s",
    num_cores=NC, num_subcores=NS)


def _deg_body(edges_hbm, out_hbm, idx_v, ones_v, zeros_v, hist):
    cid = lax.axis_index("c")
    sid = lax.axis_index("s")

    def fill(i, _):
        ones_v[pl.ds(i * LANES, LANES)] = jnp.ones((LANES,), jnp.float32)
        return _
    lax.fori_loop(0, CH // LANES, fill, None)

    def zfill(i, _):
        zeros_v[pl.ds(i * LANES, LANES)] = jnp.zeros((LANES,), jnp.float32)
        return _
    lax.fori_loop(0, STRIPE // LANES, zfill, None)

    pltpu.sync_copy(zeros_v, hist.at[pl.ds(sid * STRIPE, STRIPE)])
    plsc.subcore_barrier()

    def chunk(t, _):
        base = sid * (CH * T) + t * CH
        pltpu.sync_copy(edges_hbm.at[cid, pl.ds(base, CH)], idx_v)
        pltpu.sync_copy(ones_v, hist.at[idx_v], add=True)
        return _
    lax.fori_loop(0, T, chunk, None)

    plsc.subcore_barrier()
    pltpu.sync_copy(hist.at[pl.ds(sid * STRIPE, STRIPE)],
                    out_hbm.at[cid, pl.ds(sid * STRIPE, STRIPE)])


def _sc_degrees(edges):
    k = pl.kernel(
        _deg_body,
        out_type=jax.ShapeDtypeStruct((NC, NACC), jnp.float32),
        mesh=_mesh(),
        scratch_types=[
            pltpu.VMEM((CH,), jnp.int32),
            pltpu.VMEM((CH,), jnp.float32),
            pltpu.VMEM((STRIPE,), jnp.float32),
            pltpu.VMEM_SHARED((NACC,), jnp.float32),
        ],
    )
    return k(edges)


def _agg_body(x_hbm, src_hbm, dst_hbm, zeros_hbm, out_hbm,
              sidx, didx, rows, acc, sem):
    cid = lax.axis_index("c")
    sid = lax.axis_index("s")

    pltpu.sync_copy(zeros_hbm, acc.at[pl.ds(sid * STRIPE, STRIPE)])
    plsc.subcore_barrier()

    def chunk(t, _):
        base = sid * (CH * T) + t * CH
        pltpu.sync_copy(src_hbm.at[pl.ds(base, CH)], sidx)
        pltpu.sync_copy(dst_hbm.at[pl.ds(base, CH)], didx)
        pltpu.async_copy(x_hbm.at[cid].at[sidx], rows, sem).wait()
        pltpu.sync_copy(rows, acc.at[didx], add=True)
        return _
    lax.fori_loop(0, T, chunk, None)

    plsc.subcore_barrier()
    pltpu.sync_copy(acc.at[pl.ds(sid * STRIPE, STRIPE)],
                    out_hbm.at[cid, pl.ds(sid * STRIPE, STRIPE)])


def _sc_aggregate(x_split, src_pad, dst_pad, zeros_hbm):
    k = pl.kernel(
        _agg_body,
        out_type=jax.ShapeDtypeStruct((NC, NACC, DH), jnp.float32),
        mesh=_mesh(),
        scratch_types=[
            pltpu.VMEM((CH,), jnp.int32),
            pltpu.VMEM((CH,), jnp.int32),
            pltpu.VMEM((CH, DH), jnp.float32),
            pltpu.VMEM_SHARED((NACC, DH), jnp.float32),
            pltpu.SemaphoreType.DMA,
        ],
    )
    return k(x_split, src_pad, dst_pad, zeros_hbm)


BR = 1000
GRID = N // BR


def _scale_body(feat_ref, odeg_ref, out_ref):
    ns = lax.rsqrt(jnp.maximum(odeg_ref[...], 1.0))
    x = feat_ref[...] * ns
    out_ref[0] = x[:, :DH]
    out_ref[1] = x[:, DH:]


def _tc_scale(feat, odeg):
    return pl.pallas_call(
        _scale_body,
        grid=(GRID,),
        in_specs=[
            pl.BlockSpec((BR, D), lambda i: (i, 0)),
            pl.BlockSpec((BR, 1), lambda i: (i, 0)),
        ],
        out_specs=pl.BlockSpec((NC, BR, DH), lambda i: (0, i, 0)),
        out_shape=jax.ShapeDtypeStruct((NC, N, DH), jnp.float32),
    )(feat, odeg)


def _layer1_body(a_ref, w_ref, b_ref, odeg_ref, ideg_ref, out_ref):
    acc = (jnp.dot(a_ref[0], w_ref[:DH, :], preferred_element_type=jnp.float32)
           + jnp.dot(a_ref[1], w_ref[DH:, :], preferred_element_type=jnp.float32))
    nd = lax.rsqrt(jnp.maximum(ideg_ref[...], 1.0))
    h = jnp.maximum(acc * nd + b_ref[...], 0.0)
    ns = lax.rsqrt(jnp.maximum(odeg_ref[...], 1.0))
    x2 = h * ns
    out_ref[0] = x2[:, :DH]
    out_ref[1] = x2[:, DH:]


def _tc_layer1(a1, w1, b1, odeg, ideg):
    return pl.pallas_call(
        _layer1_body,
        grid=(GRID,),
        in_specs=[
            pl.BlockSpec((NC, BR, DH), lambda i: (0, i, 0)),
            pl.BlockSpec((D, D), lambda i: (0, 0)),
            pl.BlockSpec((1, D), lambda i: (0, 0)),
            pl.BlockSpec((BR, 1), lambda i: (i, 0)),
            pl.BlockSpec((BR, 1), lambda i: (i, 0)),
        ],
        out_specs=pl.BlockSpec((NC, BR, DH), lambda i: (0, i, 0)),
        out_shape=jax.ShapeDtypeStruct((NC, N, DH), jnp.float32),
    )(a1, w1, b1, odeg, ideg)


def _head_body(a_ref, w2_ref, b2_ref, w3_ref, b3_ref, ideg_ref, noise_ref,
               out_ref):
    a0 = a_ref[0]
    a1 = a_ref[1]
    mu = (jnp.dot(a0, w2_ref[:DH, :], preferred_element_type=jnp.float32)
          + jnp.dot(a1, w2_ref[DH:, :], preferred_element_type=jnp.float32))
    ls = (jnp.dot(a0, w3_ref[:DH, :], preferred_element_type=jnp.float32)
          + jnp.dot(a1, w3_ref[DH:, :], preferred_element_type=jnp.float32))
    nd = lax.rsqrt(jnp.maximum(ideg_ref[...], 1.0))
    mu = mu * nd + b2_ref[...]
    ls = ls * nd + b3_ref[...]
    out_ref[...] = mu + noise_ref[...] * jnp.exp(ls)


def _tc_head(a2, w2, b2, w3, b3, ideg, noise):
    return pl.pallas_call(
        _head_body,
        grid=(GRID,),
        in_specs=[
            pl.BlockSpec((NC, BR, DH), lambda i: (0, i, 0)),
            pl.BlockSpec((D, DO), lambda i: (0, 0)),
            pl.BlockSpec((1, DO), lambda i: (0, 0)),
            pl.BlockSpec((D, DO), lambda i: (0, 0)),
            pl.BlockSpec((1, DO), lambda i: (0, 0)),
            pl.BlockSpec((BR, 1), lambda i: (i, 0)),
            pl.BlockSpec((BR, DO), lambda i: (i, 0)),
        ],
        out_specs=pl.BlockSpec((BR, DO), lambda i: (i, 0)),
        out_shape=jax.ShapeDtypeStruct((N, DO), jnp.float32),
    )(a2, w2, b2, w3, b3, ideg, noise)


@jax.jit
def kernel(feat, edge_index, W1, b1, W2, b2, W3, b3, noise):
    src = edge_index[0]
    dst = edge_index[1]
    pad = EPS - E
    sentinel = jnp.full((pad,), N, jnp.int32)
    src_gather = jnp.concatenate([src, jnp.zeros((pad,), jnp.int32)])
    dst_pad = jnp.concatenate([dst, sentinel])
    edges_deg = jnp.stack([jnp.concatenate([src, sentinel]), dst_pad])
    zeros_hbm = jnp.zeros((STRIPE, DH), jnp.float32)

    degs = _sc_degrees(edges_deg)
    odeg = degs[0, :N].reshape(N, 1)
    ideg = degs[1, :N].reshape(N, 1)

    x1 = _tc_scale(feat, odeg)
    a1 = _sc_aggregate(x1, src_gather, dst_pad, zeros_hbm)
    x2 = _tc_layer1(a1, W1, b1.reshape(1, D), odeg, ideg)
    a2 = _sc_aggregate(x2, src_gather, dst_pad, zeros_hbm)
    return _tc_head(a2, W2, b2.reshape(1, DO), W3, b3.reshape(1, DO),
                    ideg, noise)


# speedup vs baseline: 3.8384x; 1.0659x over previous
import functools

import jax
import jax.numpy as jnp
from jax import lax
from jax.experimental import pallas as pl
from jax.experimental.pallas import tpu as pltpu
from jax.experimental.pallas import tpu_sc as plsc

N = 10000
E = 160000
D = 256
DH = 128
DO = 128

NC = 2
NS = 16
LANES = 16

CH = 128
T = 80
EPS = NS * CH * T
NACC = 10240
STRIPE = NACC // NS

_mesh = functools.partial(
    plsc.VectorSubcoreMesh, core_axis_name="c", subcore_axis_name="s",
    num_cores=NC, num_subcores=NS)


def _deg_body(edges_hbm, out_hbm, idx_v, ones_v, zeros_v, hist):
    cid = lax.axis_index("c")
    sid = lax.axis_index("s")

    def fill(i, _):
        ones_v[pl.ds(i * LANES, LANES)] = jnp.ones((LANES,), jnp.float32)
        return _
    lax.fori_loop(0, CH // LANES, fill, None)

    def zfill(i, _):
        zeros_v[pl.ds(i * LANES, LANES)] = jnp.zeros((LANES,), jnp.float32)
        return _
    lax.fori_loop(0, STRIPE // LANES, zfill, None)

    pltpu.sync_copy(zeros_v, hist.at[pl.ds(sid * STRIPE, STRIPE)])
    plsc.subcore_barrier()

    def chunk(t, _):
        base = sid * (CH * T) + t * CH
        pltpu.sync_copy(edges_hbm.at[cid, pl.ds(base, CH)], idx_v)
        pltpu.sync_copy(ones_v, hist.at[idx_v], add=True)
        return _
    lax.fori_loop(0, T, chunk, None)

    plsc.subcore_barrier()
    pltpu.sync_copy(hist.at[pl.ds(sid * STRIPE, STRIPE)],
                    out_hbm.at[cid, pl.ds(sid * STRIPE, STRIPE)])


def _sc_degrees(edges):
    k = pl.kernel(
        _deg_body,
        out_type=jax.ShapeDtypeStruct((NC, NACC), jnp.float32),
        mesh=_mesh(),
        scratch_types=[
            pltpu.VMEM((CH,), jnp.int32),
            pltpu.VMEM((CH,), jnp.float32),
            pltpu.VMEM((STRIPE,), jnp.float32),
            pltpu.VMEM_SHARED((NACC,), jnp.float32),
        ],
    )
    return k(edges)


def _agg_body(x_hbm, src_hbm, dst_hbm, zeros_hbm, out_hbm,
              sidx0, didx0, sidx1, didx1, rows0, rows1, acc, sem0, sem1):
    cid = lax.axis_index("c")
    sid = lax.axis_index("s")

    pltpu.sync_copy(zeros_hbm, acc.at[pl.ds(sid * STRIPE, STRIPE)])
    plsc.subcore_barrier()

    sidx = (sidx0, sidx1)
    didx = (didx0, didx1)
    rows = (rows0, rows1)
    sem = (sem0, sem1)
    ebase = sid * (CH * T)

    pltpu.sync_copy(src_hbm.at[pl.ds(ebase, CH)], sidx0)
    pltpu.sync_copy(dst_hbm.at[pl.ds(ebase, CH)], didx0)
    pltpu.async_copy(x_hbm.at[cid].at[sidx0], rows0, sem0)

    def pair(g, _):
        for b in range(2):
            t = 2 * g + b
            nb = 1 - b

            @pl.when(t + 1 < T)
            def _prefetch():
                base = ebase + (t + 1) * CH
                pltpu.sync_copy(src_hbm.at[pl.ds(base, CH)], sidx[nb])
                pltpu.sync_copy(dst_hbm.at[pl.ds(base, CH)], didx[nb])
                pltpu.async_copy(x_hbm.at[cid].at[sidx[nb]], rows[nb], sem[nb])

            pltpu.make_async_copy(x_hbm.at[cid].at[sidx[b]], rows[b],
                                  sem[b]).wait()
            pltpu.sync_copy(rows[b], acc.at[didx[b]], add=True)
        return _
    lax.fori_loop(0, T // 2, pair, None)

    plsc.subcore_barrier()
    pltpu.sync_copy(acc.at[pl.ds(sid * STRIPE, STRIPE)],
                    out_hbm.at[cid, pl.ds(sid * STRIPE, STRIPE)])


def _sc_aggregate(x_split, src_pad, dst_pad, zeros_hbm):
    k = pl.kernel(
        _agg_body,
        out_type=jax.ShapeDtypeStruct((NC, NACC, DH), jnp.float32),
        mesh=_mesh(),
        scratch_types=[
            pltpu.VMEM((CH,), jnp.int32),
            pltpu.VMEM((CH,), jnp.int32),
            pltpu.VMEM((CH,), jnp.int32),
            pltpu.VMEM((CH,), jnp.int32),
            pltpu.VMEM((CH, DH), jnp.float32),
            pltpu.VMEM((CH, DH), jnp.float32),
            pltpu.VMEM_SHARED((NACC, DH), jnp.float32),
            pltpu.SemaphoreType.DMA,
            pltpu.SemaphoreType.DMA,
        ],
    )
    return k(x_split, src_pad, dst_pad, zeros_hbm)


BR = 1000
GRID = N // BR


def _scale_body(feat_ref, odeg_ref, out_ref):
    ns = lax.rsqrt(jnp.maximum(odeg_ref[...], 1.0))
    x = feat_ref[...] * ns
    out_ref[0] = x[:, :DH]
    out_ref[1] = x[:, DH:]


def _tc_scale(feat, odeg):
    return pl.pallas_call(
        _scale_body,
        grid=(GRID,),
        in_specs=[
            pl.BlockSpec((BR, D), lambda i: (i, 0)),
            pl.BlockSpec((BR, 1), lambda i: (i, 0)),
        ],
        out_specs=pl.BlockSpec((NC, BR, DH), lambda i: (0, i, 0)),
        out_shape=jax.ShapeDtypeStruct((NC, N, DH), jnp.float32),
    )(feat, odeg)


def _layer1_body(a_ref, w_ref, b_ref, odeg_ref, ideg_ref, out_ref):
    acc = (jnp.dot(a_ref[0], w_ref[:DH, :], preferred_element_type=jnp.float32)
           + jnp.dot(a_ref[1], w_ref[DH:, :], preferred_element_type=jnp.float32))
    nd = lax.rsqrt(jnp.maximum(ideg_ref[...], 1.0))
    h = jnp.maximum(acc * nd + b_ref[...], 0.0)
    ns = lax.rsqrt(jnp.maximum(odeg_ref[...], 1.0))
    x2 = h * ns
    out_ref[0] = x2[:, :DH]
    out_ref[1] = x2[:, DH:]


def _tc_layer1(a1, w1, b1, odeg, ideg):
    return pl.pallas_call(
        _layer1_body,
        grid=(GRID,),
        in_specs=[
            pl.BlockSpec((NC, BR, DH), lambda i: (0, i, 0)),
            pl.BlockSpec((D, D), lambda i: (0, 0)),
            pl.BlockSpec((1, D), lambda i: (0, 0)),
            pl.BlockSpec((BR, 1), lambda i: (i, 0)),
            pl.BlockSpec((BR, 1), lambda i: (i, 0)),
        ],
        out_specs=pl.BlockSpec((NC, BR, DH), lambda i: (0, i, 0)),
        out_shape=jax.ShapeDtypeStruct((NC, N, DH), jnp.float32),
    )(a1, w1, b1, odeg, ideg)


def _head_body(a_ref, w2_ref, b2_ref, w3_ref, b3_ref, ideg_ref, noise_ref,
               out_ref):
    a0 = a_ref[0]
    a1 = a_ref[1]
    mu = (jnp.dot(a0, w2_ref[:DH, :], preferred_element_type=jnp.float32)
          + jnp.dot(a1, w2_ref[DH:, :], preferred_element_type=jnp.float32))
    ls = (jnp.dot(a0, w3_ref[:DH, :], preferred_element_type=jnp.float32)
          + jnp.dot(a1, w3_ref[DH:, :], preferred_element_type=jnp.float32))
    nd = lax.rsqrt(jnp.maximum(ideg_ref[...], 1.0))
    mu = mu * nd + b2_ref[...]
    ls = ls * nd + b3_ref[...]
    out_ref[...] = mu + noise_ref[...] * jnp.exp(ls)


def _tc_head(a2, w2, b2, w3, b3, ideg, noise):
    return pl.pallas_call(
        _head_body,
        grid=(GRID,),
        in_specs=[
            pl.BlockSpec((NC, BR, DH), lambda i: (0, i, 0)),
            pl.BlockSpec((D, DO), lambda i: (0, 0)),
            pl.BlockSpec((1, DO), lambda i: (0, 0)),
            pl.BlockSpec((D, DO), lambda i: (0, 0)),
            pl.BlockSpec((1, DO), lambda i: (0, 0)),
            pl.BlockSpec((BR, 1), lambda i: (i, 0)),
            pl.BlockSpec((BR, DO), lambda i: (i, 0)),
        ],
        out_specs=pl.BlockSpec((BR, DO), lambda i: (i, 0)),
        out_shape=jax.ShapeDtypeStruct((N, DO), jnp.float32),
    )(a2, w2, b2, w3, b3, ideg, noise)


@jax.jit
def kernel(feat, edge_index, W1, b1, W2, b2, W3, b3, noise):
    src = edge_index[0]
    dst = edge_index[1]
    pad = EPS - E
    sentinel = jnp.full((pad,), N, jnp.int32)
    src_gather = jnp.concatenate([src, jnp.zeros((pad,), jnp.int32)])
    dst_pad = jnp.concatenate([dst, sentinel])
    edges_deg = jnp.stack([jnp.concatenate([src, sentinel]), dst_pad])
    zeros_hbm = jnp.zeros((STRIPE, DH), jnp.float32)

    degs = _sc_degrees(edges_deg)
    odeg = degs[0, :N].reshape(N, 1)
    ideg = degs[1, :N].reshape(N, 1)

    x1 = _tc_scale(feat, odeg)
    a1 = _sc_aggregate(x1, src_gather, dst_pad, zeros_hbm)
    x2 = _tc_layer1(a1, W1, b1.reshape(1, D), odeg, ideg)
    a2 = _sc_aggregate(x2, src_gather, dst_pad, zeros_hbm)
    return _tc_head(a2, W2, b2.reshape(1, DO), W3, b3.reshape(1, DO),
                    ideg, noise)
